# baseline (speedup 1.0000x reference)
import functools

import jax
import jax.numpy as jnp
from jax import lax
from jax.experimental import pallas as pl
from jax.experimental.pallas import tpu as pltpu
from jax.experimental.pallas import tpu_sc as plsc

N_NODE = 50000
N_EDGE = 800000
EMB = 64
QW = 16
NQ = EMB // QW

NC = 2
NS = 16
B = 128
BPT = 400
NB = 4
EP = NS * BPT * B
ACC_R = 50048
STRIPE = ACC_R // NS


def _fill(buf, nrows, width, value):
    vec = jnp.full((16,), value, jnp.float32)

    def fv(i, carry):
        for j in range(width // 16):
            buf[i, pl.ds(j * 16, 16)] = vec
        return carry

    lax.fori_loop(0, nrows, fv, 0)


def _zero_acc(acc, buf, s):
    _fill(buf, B, QW, 0.0)

    def zs(k, carry):
        pltpu.sync_copy(buf, acc.at[pl.ds(s * STRIPE + k * B, B)])
        return carry

    lax.fori_loop(0, STRIPE // B, zs, 0)
    rem = STRIPE - (STRIPE // B) * B
    pltpu.sync_copy(buf.at[pl.ds(0, rem)],
                    acc.at[pl.ds(s * STRIPE + (STRIPE // B) * B, rem)])


def _relation_round(table, gidx, q, sidx, out, ch, c, s,
                    g_i, s_i, rows, gsem, ssem, ig, isx, acc):
    nch = BPT // ch
    _zero_acc(acc, rows[0], s)
    plsc.subcore_barrier()
    pltpu.async_copy(gidx.at[q, pl.ds(s * BPT, ch)], g_i.at[0], ig)
    pltpu.async_copy(sidx.at[pl.ds(s * BPT, ch)], s_i.at[0], isx)

    def chunk(k, carry):
        cur = lax.rem(k, 2)
        nxt = 1 - cur
        pltpu.make_async_copy(gidx.at[q, pl.ds(0, ch)],
                              g_i.at[cur], ig).wait()
        pltpu.make_async_copy(sidx.at[pl.ds(0, ch)], s_i.at[cur], isx).wait()

        @pl.when(k + 1 < nch)
        def _():
            off = s * BPT + (k + 1) * ch
            pltpu.async_copy(gidx.at[q, pl.ds(off, ch)], g_i.at[nxt], ig)
            pltpu.async_copy(sidx.at[pl.ds(off, ch)], s_i.at[nxt], isx)

        gd = [None] * ch
        sd = [None] * ch

        def scat(p):
            gd[p].wait()
            sd[p] = pltpu.async_copy(rows[p % NB], acc.at[s_i.at[cur, p]],
                                     ssem[p % NB], add=True)

        for p in range(ch):
            if p >= NB:
                sd[p - NB].wait()
            gd[p] = pltpu.async_copy(table.at[g_i.at[cur, p]],
                                     rows[p % NB], gsem[p % NB])
            if p >= 2:
                scat(p - 2)
        for p in range(ch - 2, ch):
            scat(p)
        for p in range(ch - NB, ch):
            sd[p].wait()
        return carry

    lax.fori_loop(0, nch, chunk, 0)
    plsc.subcore_barrier()
    pltpu.sync_copy(acc.at[pl.ds(s * STRIPE, STRIPE)],
                    out.at[q, pl.ds(s * STRIPE, STRIPE)])


def _hist(cidx, out, ch, c, s, s_i, ones, ssem, isx, acc):
    nch = BPT // ch
    _zero_acc(acc, ones, s)
    _fill(ones, B, QW, 1.0)
    plsc.subcore_barrier()
    pltpu.async_copy(cidx.at[c, pl.ds(s * BPT, ch)], s_i.at[0], isx)

    def chunk(k, carry):
        cur = lax.rem(k, 2)
        nxt = 1 - cur
        pltpu.make_async_copy(cidx.at[c, pl.ds(0, ch)],
                              s_i.at[cur], isx).wait()

        @pl.when(k + 1 < nch)
        def _():
            off = s * BPT + (k + 1) * ch
            pltpu.async_copy(cidx.at[c, pl.ds(off, ch)], s_i.at[nxt], isx)

        sd = [None] * ch
        for p in range(ch):
            if p >= NB:
                sd[p - NB].wait()
            sd[p] = pltpu.async_copy(ones, acc.at[s_i.at[cur, p]],
                                     ssem[p % NB], add=True)
        for p in range(ch - NB, ch):
            sd[p].wait()
        return carry

    lax.fori_loop(0, nch, chunk, 0)
    plsc.subcore_barrier()
    pltpu.sync_copy(acc.at[pl.ds(s * STRIPE, STRIPE)],
                    out.at[c, pl.ds(s * STRIPE, STRIPE)])


CH_A = 16
CH_B = 8


def _layer1_body(tab_c, tab_v, g1, g2, s1, s2, cidx, out_v, out_c, out_n,
                 g_i, s_i, r0, r1, r2, r3,
                 gs0, gs1, gs2, gs3, ss0, ss1, ss2, ss3, ig, isx, acc):
    c = lax.axis_index("c")
    s = lax.axis_index("s")
    rows = [r0, r1, r2, r3]
    gsem = [gs0, gs1, gs2, gs3]
    ssem = [ss0, ss1, ss2, ss3]
    for r in range(2):
        _relation_round(tab_c, g1, 2 * c + r, s1, out_v, CH_A, c, s,
                        g_i, s_i, rows, gsem, ssem, ig, isx, acc)
    for r in range(2):
        _relation_round(tab_v, g2, 2 * c + r, s2, out_c, CH_A, c, s,
                        g_i, s_i, rows, gsem, ssem, ig, isx, acc)
    _hist(cidx, out_n, CH_A, c, s, s_i, r0, ssem, isx, acc)


def _layer2_body(tab_c, g1, s1, out_v,
                 g_i, s_i, r0, r1, r2, r3,
                 gs0, gs1, gs2, gs3, ss0, ss1, ss2, ss3, ig, isx, acc):
    c = lax.axis_index("c")
    s = lax.axis_index("s")
    rows = [r0, r1, r2, r3]
    gsem = [gs0, gs1, gs2, gs3]
    ssem = [ss0, ss1, ss2, ss3]
    for r in range(2):
        _relation_round(tab_c, g1, 2 * c + r, s1, out_v, CH_B, c, s,
                        g_i, s_i, rows, gsem, ssem, ig, isx, acc)


def _sc_scratch(ch):
    return [
        pltpu.VMEM((2, ch, B), jnp.int32),
        pltpu.VMEM((2, ch, B), jnp.int32),
        pltpu.VMEM((B, QW), jnp.float32),
        pltpu.VMEM((B, QW), jnp.float32),
        pltpu.VMEM((B, QW), jnp.float32),
        pltpu.VMEM((B, QW), jnp.float32),
    ] + [pltpu.SemaphoreType.DMA] * 10 + [
        pltpu.VMEM_SHARED((ACC_R, QW), jnp.float32),
    ]


@functools.cache
def _get_layer1():
    mesh = plsc.VectorSubcoreMesh(core_axis_name="c", subcore_axis_name="s",
                                  num_cores=NC, num_subcores=NS)
    sum_ty = jax.ShapeDtypeStruct((NQ, ACC_R, QW), jnp.float32)
    cnt_ty = jax.ShapeDtypeStruct((NC, ACC_R, QW), jnp.float32)
    return functools.partial(
        pl.kernel,
        out_type=[sum_ty, sum_ty, cnt_ty],
        mesh=mesh,
        scratch_types=_sc_scratch(CH_A),
        compiler_params=pltpu.CompilerParams(use_tc_tiling_on_sc=False),
    )(_layer1_body)


@functools.cache
def _get_layer2():
    mesh = plsc.VectorSubcoreMesh(core_axis_name="c", subcore_axis_name="s",
                                  num_cores=NC, num_subcores=NS)
    sum_ty = jax.ShapeDtypeStruct((NQ, ACC_R, QW), jnp.float32)
    return functools.partial(
        pl.kernel,
        out_type=sum_ty,
        mesh=mesh,
        scratch_types=_sc_scratch(CH_B),
        compiler_params=pltpu.CompilerParams(use_tc_tiling_on_sc=False),
    )(_layer2_body)


def _sc_layer1(*args):
    return _get_layer1()(*args)


def _sc_layer2(*args):
    return _get_layer2()(*args)


N_PAD = ACC_R
RB = 2176
GRID = N_PAD // RB
PR = RB // 8

def _bcast(i):
    return (0, 0)


def _embed_body(cx, vx, bi, cw1, cb1, cw2, cb2, vw1, vb1, vw2, vb2, bw,
                ch_o, vh_o):
    f32 = jnp.float32
    ch = jnp.maximum(jnp.dot(cx[...], cw1[...], preferred_element_type=f32)
                     + cb1[...], 0.0)
    ch = jnp.maximum(jnp.dot(ch, cw2[...], preferred_element_type=f32)
                     + cb2[...], 0.0)
    vh = jnp.maximum(jnp.dot(vx[...], vw1[...], preferred_element_type=f32)
                     + vb1[...], 0.0)
    vh = jnp.maximum(jnp.dot(vh, vw2[...], preferred_element_type=f32)
                     + vb2[...], 0.0)
    vh = vh + bi[...] * bw[...]
    ch_o[...] = ch
    vh_o[...] = vh


def _make_embed():
    wspec = lambda shp: pl.BlockSpec(shp, _bcast)
    return pl.pallas_call(
        _embed_body,
        grid=(GRID,),
        in_specs=[
            pl.BlockSpec((RB, 8), lambda i: (i, 0)),
            pl.BlockSpec((RB, 24), lambda i: (i, 0)),
            pl.BlockSpec((RB, 1), lambda i: (i, 0)),
            wspec((8, EMB)), wspec((1, EMB)),
            wspec((EMB, EMB)), wspec((1, EMB)),
            wspec((24, EMB)), wspec((1, EMB)),
            wspec((EMB, EMB)), wspec((1, EMB)),
            wspec((1, EMB)),
        ],
        out_specs=[
            pl.BlockSpec((RB, EMB), lambda i: (i, 0)),
            pl.BlockSpec((RB, EMB), lambda i: (i, 0)),
        ],
        out_shape=[
            jax.ShapeDtypeStruct((N_PAD, EMB), jnp.float32),
            jax.ShapeDtypeStruct((N_PAD, EMB), jnp.float32),
        ],
    )


def _upd_side_body(sm, cnts, h_r, wl, bl, wr, h_o):
    f32 = jnp.float32
    sm_a = sm[...]
    cn_a = cnts[...]
    mean = (jnp.concatenate([sm_a[q] for q in range(NQ)], axis=1)
            / jnp.maximum(cn_a[0][:, 0:1], 1.0))
    nh = (jnp.dot(mean, wl[...], preferred_element_type=f32) + bl[...]
          + jnp.dot(h_r[...], wr[...], preferred_element_type=f32))
    h_o[...] = jnp.maximum(nh, 0.0)


def _make_upd_side(cslot):
    wspec = lambda shp: pl.BlockSpec(shp, _bcast)
    return pl.pallas_call(
        _upd_side_body,
        grid=(GRID,),
        in_specs=[
            pl.BlockSpec((NQ, RB, QW), lambda i: (0, i, 0)),
            pl.BlockSpec((1, RB, QW), lambda i: (cslot, i, 0)),
            pl.BlockSpec((RB, EMB), lambda i: (i, 0)),
            wspec((EMB, EMB)), wspec((1, EMB)), wspec((EMB, EMB)),
        ],
        out_specs=pl.BlockSpec((RB, EMB), lambda i: (i, 0)),
        out_shape=jax.ShapeDtypeStruct((N_PAD, EMB), jnp.float32),
    )


_embed_call = _make_embed()
_upd_var_call = _make_upd_side(0)
_upd_cons_call = _make_upd_side(1)


def kernel(cons_x, var_x, edge_index, edge_attr, break_indicator,
           cons_shift, cons_scale, cons_W1, cons_b1, cons_W2, cons_b2,
           var_shift, var_scale, var_W1, var_b1, var_W2, var_b2,
           edge_shift, edge_scale, break_W, lin_l_W, lin_l_b, lin_r_W):
    del edge_attr, edge_shift, edge_scale

    cw1 = cons_scale[:, None] * cons_W1
    cb1 = (cons_b1 + (cons_shift * cons_scale) @ cons_W1)[None, :]
    vw1 = var_scale[:, None] * var_W1
    vb1 = (var_b1 + (var_shift * var_scale) @ var_W1)[None, :]
    rpad = N_PAD - N_NODE
    cx = jnp.pad(cons_x, ((0, rpad), (0, 3)))
    vx = jnp.pad(var_x, ((0, rpad), (0, 5)))
    bi = jnp.pad(break_indicator, ((0, rpad), (0, 0)))
    cw1 = jnp.pad(cw1, ((0, 3), (0, 0)))
    vw1 = jnp.pad(vw1, ((0, 5), (0, 0)))

    src = edge_index[0].astype(jnp.int32)
    dst = edge_index[1].astype(jnp.int32)
    padn = EP - N_EDGE
    src_g = jnp.pad(src, (0, padn))
    dst_g = jnp.pad(dst, (0, padn))
    src_s = jnp.pad(src, (0, padn), constant_values=N_NODE)
    dst_s = jnp.pad(dst, (0, padn), constant_values=N_NODE)

    def gidx_of(x):
        return jnp.stack([NQ * x + q for q in range(NQ)]).reshape(
            NQ, NS * BPT, B)

    g_rel1 = gidx_of(src_g)
    g_rel2 = gidx_of(dst_g)
    s_rel1 = dst_s.reshape(NS * BPT, B)
    s_rel2 = src_s.reshape(NS * BPT, B)
    c_idx = jnp.stack([dst_s, src_s]).reshape(NC, NS * BPT, B)

    ch0, vh0 = _embed_call(cx, vx, bi, cw1, cb1,
                           cons_W2, cons_b2[None, :], vw1, vb1,
                           var_W2, var_b2[None, :], break_W)

    sv1, sc1, cn = _sc_layer1(ch0.reshape(NQ * N_PAD, QW),
                              vh0.reshape(NQ * N_PAD, QW),
                              g_rel1, g_rel2, s_rel1, s_rel2, c_idx)
    ch1 = _upd_cons_call(sc1, cn, ch0,
                         lin_l_W[0, 1], lin_l_b[0, 1][None, :],
                         lin_r_W[0, 1])

    sv2 = _sc_layer2(ch1.reshape(NQ * N_PAD, QW), g_rel1, s_rel1)
    vh1 = _upd_var_call(sv1, cn, vh0,
                        lin_l_W[0, 0], lin_l_b[0, 0][None, :],
                        lin_r_W[0, 0])
    vh2 = _upd_var_call(sv2, cn, vh1,
                        lin_l_W[1, 0], lin_l_b[1, 0][None, :],
                        lin_r_W[1, 0])
    return vh2[:N_NODE]


# speedup vs baseline: 1.0102x; 1.0102x over previous
import functools

import jax
import jax.numpy as jnp
from jax import lax
from jax.experimental import pallas as pl
from jax.experimental.pallas import tpu as pltpu
from jax.experimental.pallas import tpu_sc as plsc

N_NODE = 50000
N_EDGE = 800000
EMB = 64
QW = 16
NQ = EMB // QW

NC = 2
NS = 16
B = 128
BPT = 400
NB = 4
EP = NS * BPT * B
ACC_R = 50048
STRIPE = ACC_R // NS


def _fill(buf, nrows, width, value):
    vec = jnp.full((16,), value, jnp.float32)

    def fv(i, carry):
        for j in range(width // 16):
            buf[i, pl.ds(j * 16, 16)] = vec
        return carry

    lax.fori_loop(0, nrows, fv, 0)


def _zero_acc(acc, buf, s):
    _fill(buf, B, QW, 0.0)

    def zs(k, carry):
        pltpu.sync_copy(buf, acc.at[pl.ds(s * STRIPE + k * B, B)])
        return carry

    lax.fori_loop(0, STRIPE // B, zs, 0)
    rem = STRIPE - (STRIPE // B) * B
    pltpu.sync_copy(buf.at[pl.ds(0, rem)],
                    acc.at[pl.ds(s * STRIPE + (STRIPE // B) * B, rem)])


def _relation_round(table, gidx, q, sidx, out, ch, c, s,
                    g_i, s_i, rows, gsem, ssem, ig, isx, acc):
    nch = BPT // ch
    _zero_acc(acc, rows[0], s)
    plsc.subcore_barrier()
    pltpu.async_copy(gidx.at[q, pl.ds(s * BPT, ch)], g_i.at[0], ig)
    pltpu.async_copy(sidx.at[pl.ds(s * BPT, ch)], s_i.at[0], isx)

    def chunk(k, carry):
        cur = lax.rem(k, 2)
        nxt = 1 - cur
        pltpu.make_async_copy(gidx.at[q, pl.ds(0, ch)],
                              g_i.at[cur], ig).wait()
        pltpu.make_async_copy(sidx.at[pl.ds(0, ch)], s_i.at[cur], isx).wait()

        @pl.when(k + 1 < nch)
        def _():
            off = s * BPT + (k + 1) * ch
            pltpu.async_copy(gidx.at[q, pl.ds(off, ch)], g_i.at[nxt], ig)
            pltpu.async_copy(sidx.at[pl.ds(off, ch)], s_i.at[nxt], isx)

        gd = [None] * ch
        sd = [None] * ch

        def scat(p):
            gd[p].wait()
            sd[p] = pltpu.async_copy(rows[p % NB], acc.at[s_i.at[cur, p]],
                                     ssem[p % NB], add=True)

        for p in range(ch):
            if p >= NB:
                sd[p - NB].wait()
            gd[p] = pltpu.async_copy(table.at[g_i.at[cur, p]],
                                     rows[p % NB], gsem[p % NB])
            if p >= 2:
                scat(p - 2)
        for p in range(ch - 2, ch):
            scat(p)
        for p in range(ch - NB, ch):
            sd[p].wait()
        return carry

    lax.fori_loop(0, nch, chunk, 0)
    plsc.subcore_barrier()
    pltpu.sync_copy(acc.at[pl.ds(s * STRIPE, STRIPE)],
                    out.at[q, pl.ds(s * STRIPE, STRIPE)])


def _hist(cidx, out, ch, c, s, s_i, ones, ssem, isx, acc):
    nch = BPT // ch
    _zero_acc(acc, ones, s)
    _fill(ones, B, QW, 1.0)
    plsc.subcore_barrier()
    pltpu.async_copy(cidx.at[c, pl.ds(s * BPT, ch)], s_i.at[0], isx)

    def chunk(k, carry):
        cur = lax.rem(k, 2)
        nxt = 1 - cur
        pltpu.make_async_copy(cidx.at[c, pl.ds(0, ch)],
                              s_i.at[cur], isx).wait()

        @pl.when(k + 1 < nch)
        def _():
            off = s * BPT + (k + 1) * ch
            pltpu.async_copy(cidx.at[c, pl.ds(off, ch)], s_i.at[nxt], isx)

        sd = [None] * ch
        for p in range(ch):
            if p >= NB:
                sd[p - NB].wait()
            sd[p] = pltpu.async_copy(ones, acc.at[s_i.at[cur, p]],
                                     ssem[p % NB], add=True)
        for p in range(ch - NB, ch):
            sd[p].wait()
        return carry

    lax.fori_loop(0, nch, chunk, 0)
    plsc.subcore_barrier()
    pltpu.sync_copy(acc.at[pl.ds(s * STRIPE, STRIPE)],
                    out.at[c, pl.ds(s * STRIPE, STRIPE)])


CH_A = 16
CH_B = 8


def _layer1_body(tab_c, tab_v, g1, g2, s1, s2, cidx, out_v, out_c, out_n,
                 g_i, s_i, r0, r1, r2, r3,
                 gs0, gs1, gs2, gs3, ss0, ss1, ss2, ss3, ig, isx, acc):
    c = lax.axis_index("c")
    s = lax.axis_index("s")
    rows = [r0, r1, r2, r3]
    gsem = [gs0, gs1, gs2, gs3]
    ssem = [ss0, ss1, ss2, ss3]
    for r in range(2):
        _relation_round(tab_c, g1, 2 * c + r, s1, out_v, CH_A, c, s,
                        g_i, s_i, rows, gsem, ssem, ig, isx, acc)
    for r in range(2):
        _relation_round(tab_v, g2, 2 * c + r, s2, out_c, CH_A, c, s,
                        g_i, s_i, rows, gsem, ssem, ig, isx, acc)
    _hist(cidx, out_n, CH_A, c, s, s_i, r0, ssem, isx, acc)


def _layer2_body(tab_c, g1, s1, out_v,
                 g_i, s_i, r0, r1, r2, r3,
                 gs0, gs1, gs2, gs3, ss0, ss1, ss2, ss3, ig, isx, acc):
    c = lax.axis_index("c")
    s = lax.axis_index("s")
    rows = [r0, r1, r2, r3]
    gsem = [gs0, gs1, gs2, gs3]
    ssem = [ss0, ss1, ss2, ss3]
    for r in range(2):
        _relation_round(tab_c, g1, 2 * c + r, s1, out_v, CH_B, c, s,
                        g_i, s_i, rows, gsem, ssem, ig, isx, acc)


def _sc_scratch(ch):
    return [
        pltpu.VMEM((2, ch, B), jnp.int32),
        pltpu.VMEM((2, ch, B), jnp.int32),
        pltpu.VMEM((B, QW), jnp.float32),
        pltpu.VMEM((B, QW), jnp.float32),
        pltpu.VMEM((B, QW), jnp.float32),
        pltpu.VMEM((B, QW), jnp.float32),
    ] + [pltpu.SemaphoreType.DMA] * 10 + [
        pltpu.VMEM_SHARED((ACC_R, QW), jnp.float32),
    ]


@functools.cache
def _get_layer1():
    mesh = plsc.VectorSubcoreMesh(core_axis_name="c", subcore_axis_name="s",
                                  num_cores=NC, num_subcores=NS)
    sum_ty = jax.ShapeDtypeStruct((NQ, ACC_R, QW), jnp.float32)
    cnt_ty = jax.ShapeDtypeStruct((NC, ACC_R, QW), jnp.float32)
    return functools.partial(
        pl.kernel,
        out_type=[sum_ty, sum_ty, cnt_ty],
        mesh=mesh,
        scratch_types=_sc_scratch(CH_A),
        compiler_params=pltpu.CompilerParams(use_tc_tiling_on_sc=False),
    )(_layer1_body)


@functools.cache
def _get_layer2():
    mesh = plsc.VectorSubcoreMesh(core_axis_name="c", subcore_axis_name="s",
                                  num_cores=NC, num_subcores=NS)
    sum_ty = jax.ShapeDtypeStruct((NQ, ACC_R, QW), jnp.float32)
    return functools.partial(
        pl.kernel,
        out_type=sum_ty,
        mesh=mesh,
        scratch_types=_sc_scratch(CH_B),
        compiler_params=pltpu.CompilerParams(use_tc_tiling_on_sc=False),
    )(_layer2_body)


def _sc_layer1(*args):
    return _get_layer1()(*args)


def _sc_layer2(*args):
    return _get_layer2()(*args)


RB = 2000
GRID = N_NODE // RB

def _bcast(i):
    return (0, 0)


def _embed_body(cx, vx, bi, cw1, cb1, cw2, cb2, vw1, vb1, vw2, vb2, bw,
                ch_o, vh_o):
    f32 = jnp.float32
    ch = jnp.maximum(jnp.dot(cx[...], cw1[...], preferred_element_type=f32)
                     + cb1[...], 0.0)
    ch = jnp.maximum(jnp.dot(ch, cw2[...], preferred_element_type=f32)
                     + cb2[...], 0.0)
    vh = jnp.maximum(jnp.dot(vx[...], vw1[...], preferred_element_type=f32)
                     + vb1[...], 0.0)
    vh = jnp.maximum(jnp.dot(vh, vw2[...], preferred_element_type=f32)
                     + vb2[...], 0.0)
    vh = vh + bi[...] * bw[...]
    ch_o[...] = ch
    vh_o[...] = vh


def _make_embed():
    wspec = lambda shp: pl.BlockSpec(shp, _bcast)
    return pl.pallas_call(
        _embed_body,
        grid=(GRID,),
        in_specs=[
            pl.BlockSpec((RB, 8), lambda i: (i, 0)),
            pl.BlockSpec((RB, 24), lambda i: (i, 0)),
            pl.BlockSpec((RB, 1), lambda i: (i, 0)),
            wspec((8, EMB)), wspec((1, EMB)),
            wspec((EMB, EMB)), wspec((1, EMB)),
            wspec((24, EMB)), wspec((1, EMB)),
            wspec((EMB, EMB)), wspec((1, EMB)),
            wspec((1, EMB)),
        ],
        out_specs=[
            pl.BlockSpec((RB, EMB), lambda i: (i, 0)),
            pl.BlockSpec((RB, EMB), lambda i: (i, 0)),
        ],
        out_shape=[
            jax.ShapeDtypeStruct((N_NODE, EMB), jnp.float32),
            jax.ShapeDtypeStruct((N_NODE, EMB), jnp.float32),
        ],
    )


def _upd_side_body(sm, cnts, h_r, wl, bl, wr, h_o):
    f32 = jnp.float32
    sm_a = sm[...]
    cn_a = cnts[...]
    mean = (jnp.concatenate([sm_a[q] for q in range(NQ)], axis=1)
            / jnp.maximum(cn_a[0][:, 0:1], 1.0))
    nh = (jnp.dot(mean, wl[...], preferred_element_type=f32) + bl[...]
          + jnp.dot(h_r[...], wr[...], preferred_element_type=f32))
    h_o[...] = jnp.maximum(nh, 0.0)


def _make_upd_side(cslot):
    wspec = lambda shp: pl.BlockSpec(shp, _bcast)
    return pl.pallas_call(
        _upd_side_body,
        grid=(GRID,),
        in_specs=[
            pl.BlockSpec((NQ, RB, QW), lambda i: (0, i, 0)),
            pl.BlockSpec((1, RB, QW), lambda i: (cslot, i, 0)),
            pl.BlockSpec((RB, EMB), lambda i: (i, 0)),
            wspec((EMB, EMB)), wspec((1, EMB)), wspec((EMB, EMB)),
        ],
        out_specs=pl.BlockSpec((RB, EMB), lambda i: (i, 0)),
        out_shape=jax.ShapeDtypeStruct((N_NODE, EMB), jnp.float32),
    )


_embed_call = _make_embed()
_upd_var_call = _make_upd_side(0)
_upd_cons_call = _make_upd_side(1)


def kernel(cons_x, var_x, edge_index, edge_attr, break_indicator,
           cons_shift, cons_scale, cons_W1, cons_b1, cons_W2, cons_b2,
           var_shift, var_scale, var_W1, var_b1, var_W2, var_b2,
           edge_shift, edge_scale, break_W, lin_l_W, lin_l_b, lin_r_W):
    del edge_attr, edge_shift, edge_scale

    cw1 = cons_scale[:, None] * cons_W1
    cb1 = (cons_b1 + (cons_shift * cons_scale) @ cons_W1)[None, :]
    vw1 = var_scale[:, None] * var_W1
    vb1 = (var_b1 + (var_shift * var_scale) @ var_W1)[None, :]
    cx = jnp.pad(cons_x, ((0, 0), (0, 3)))
    vx = jnp.pad(var_x, ((0, 0), (0, 5)))
    cw1 = jnp.pad(cw1, ((0, 3), (0, 0)))
    vw1 = jnp.pad(vw1, ((0, 5), (0, 0)))

    src = edge_index[0].astype(jnp.int32)
    dst = edge_index[1].astype(jnp.int32)
    padn = EP - N_EDGE
    src_g = jnp.pad(src, (0, padn))
    dst_g = jnp.pad(dst, (0, padn))
    src_s = jnp.pad(src, (0, padn), constant_values=N_NODE)
    dst_s = jnp.pad(dst, (0, padn), constant_values=N_NODE)

    def gidx_of(x):
        return jnp.stack([NQ * x + q for q in range(NQ)]).reshape(
            NQ, NS * BPT, B)

    g_rel1 = gidx_of(src_g)
    g_rel2 = gidx_of(dst_g)
    s_rel1 = dst_s.reshape(NS * BPT, B)
    s_rel2 = src_s.reshape(NS * BPT, B)
    c_idx = jnp.stack([dst_s, src_s]).reshape(NC, NS * BPT, B)

    ch0, vh0 = _embed_call(cx, vx, break_indicator, cw1, cb1,
                           cons_W2, cons_b2[None, :], vw1, vb1,
                           var_W2, var_b2[None, :], break_W)

    sv1, sc1, cn = _sc_layer1(ch0.reshape(NQ * N_NODE, QW),
                              vh0.reshape(NQ * N_NODE, QW),
                              g_rel1, g_rel2, s_rel1, s_rel2, c_idx)
    ch1 = _upd_cons_call(sc1, cn, ch0,
                         lin_l_W[0, 1], lin_l_b[0, 1][None, :],
                         lin_r_W[0, 1])

    sv2 = _sc_layer2(ch1.reshape(NQ * N_NODE, QW), g_rel1, s_rel1)
    vh1 = _upd_var_call(sv1, cn, vh0,
                        lin_l_W[0, 0], lin_l_b[0, 0][None, :],
                        lin_r_W[0, 0])
    vh2 = _upd_var_call(sv2, cn, vh1,
                        lin_l_W[1, 0], lin_l_b[1, 0][None, :],
                        lin_r_W[1, 0])
    return vh2
